# baseline (speedup 1.0000x reference)
import jax
import jax.numpy as jnp
from jax import lax
from jax.experimental import pallas as pl
from jax.experimental.pallas import tpu as pltpu
from jax.experimental.pallas import tpu_sc as plsc

_B = 8
_C = 96
_S = 224 * 224
_N = _C * _S
_K = int(round(0.1 * _N))

_NCORES = 2
_NSUB = 16
_ROWS_PER_CORE = _B // _NCORES
_TILES_PER_ROW = _NSUB // _ROWS_PER_CORE
_CH_PER_TILE = _C // _TILES_PER_ROW
_CHUNK = _S // 4
_CHUNKS_PER_TILE = _CH_PER_TILE * 4
_VREGS_PER_CHUNK = _CHUNK // 16

_MIN_I32 = -(2 ** 31)
_PASSES = ((21, 11), (10, 11), (0, 10))


def _sc_body(x_hbm, bf_hbm, out_hbm, bf_v, stage_v, hist_v, tvec_v, merged_sh):
    c = lax.axis_index("c")
    s = lax.axis_index("s")
    local_row = s // _TILES_PER_ROW
    grp = s % _TILES_PER_ROW
    row = c * _ROWS_PER_CORE + local_row
    lane = lax.iota(jnp.int32, 16)
    ones = jnp.ones((16,), jnp.int32)
    zeros16 = jnp.zeros((16,), jnp.int32)

    pltpu.sync_copy(bf_hbm, bf_v)

    rem = jnp.int32(_K)
    prefix = jnp.int32(0)

    for shift, width in _PASSES:
        nbins = 1 << width
        nchunks = nbins // 16
        first = shift == _PASSES[0][0]

        def zrow(l, _):
            def zcol(cb, _):
                hist_v[l, pl.ds(cb * 16, 16)] = zeros16
                return 0
            lax.fori_loop(0, 2048 // 16, zcol, 0)
            return 0
        lax.fori_loop(0, 16, zrow, 0)
        pltpu.sync_copy(hist_v.at[pl.ds(0, 4)], merged_sh.at[pl.ds(s * 4, 4)])
        plsc.subcore_barrier()

        shift_vec = jnp.full((16,), shift, jnp.int32)
        binmask_vec = jnp.full((16,), nbins - 1, jnp.int32)
        pshift_vec = jnp.full((16,), shift + width, jnp.int32)
        prefix_vec = jnp.full((16,), prefix, jnp.int32)

        def chunk_body(i, _):
            ch = grp * _CH_PER_TILE + i // 4
            q = i % 4
            bfv = bf_v[ch, :]
            pltpu.sync_copy(x_hbm.at[row, ch, pl.ds(q * _CHUNK, _CHUNK)],
                            stage_v)

            def vbody(j, _):
                v = stage_v[pl.ds(j * 16, 16)]
                boosted = v * bfv
                bits = lax.bitcast_convert_type(boosted, jnp.int32)
                neg = bits >> 31
                key = bits ^ (neg | _MIN_I32)
                bin_ = lax.shift_right_logical(key, shift_vec) & binmask_vec
                if first:
                    plsc.addupdate_scatter(hist_v, [lane, bin_], ones)
                else:
                    mk = lax.shift_right_logical(key, pshift_vec) == prefix_vec
                    plsc.addupdate_scatter(hist_v, [lane, bin_], ones, mask=mk)
                return 0
            lax.fori_loop(0, _VREGS_PER_CHUNK, vbody, 0)
            return 0
        lax.fori_loop(0, _CHUNKS_PER_TILE, chunk_body, 0)

        pltpu.sync_copy(hist_v, merged_sh.at[local_row * 16 + lane], add=True)
        plsc.subcore_barrier()
        pltpu.sync_copy(merged_sh.at[pl.ds(local_row * 16, 16)], hist_v)
        plsc.subcore_barrier()

        rem_vec = jnp.full((16,), rem, jnp.int32)

        def scan_body(i, carry):
            cum, found, b_sel, rem_next = carry
            cidx = nchunks - 1 - i
            cnt = hist_v[0, pl.ds(cidx * 16, 16)]
            for l in range(1, 16):
                cnt = cnt + hist_v[l, pl.ds(cidx * 16, 16)]
            rv = lax.rev(cnt, (0,))
            cs = plsc.cumsum(rv)
            tot = cs + cum
            crossed = tot >= rem_vec
            chunk_total = jnp.sum(cnt)
            cum_new = cum + chunk_total
            chunk_crossed = cum_new >= rem
            ffs = plsc.all_reduce_ffs(crossed)
            l_scalar = jnp.max(ffs)
            l_vec = jnp.full((16,), l_scalar, jnp.int32)
            selm = lane == l_vec
            cum_before = jnp.sum(jnp.where(selm, tot - rv, 0))
            bin_found = cidx * 16 + (15 - l_scalar)
            take = jnp.logical_and(chunk_crossed, jnp.logical_not(found))
            b_sel = jnp.where(take, bin_found, b_sel)
            rem_next = jnp.where(take, rem - cum_before, rem_next)
            found = jnp.logical_or(found, chunk_crossed)
            return (cum_new, found, b_sel, rem_next)

        init = (jnp.int32(0), jnp.bool_(False), jnp.int32(0), rem)
        _, _, b_sel, rem = lax.fori_loop(0, nchunks, scan_body, init)
        prefix = (prefix << width) | b_sel

    @pl.when(grp == 0)
    def _():
        tvec_v[...] = jnp.full((16,), prefix, jnp.int32)
        pltpu.sync_copy(tvec_v, out_hbm.at[row])


def _sc_select(x3, bf_pad):
    mesh = plsc.VectorSubcoreMesh(core_axis_name="c", subcore_axis_name="s")
    f = pl.kernel(
        _sc_body,
        out_type=jax.ShapeDtypeStruct((_B, 16), jnp.int32),
        mesh=mesh,
        scratch_types=[
            pltpu.VMEM((_C, 16), jnp.float32),
            pltpu.VMEM((_CHUNK,), jnp.float32),
            pltpu.VMEM((16, 2048), jnp.int32),
            pltpu.VMEM((16,), jnp.int32),
            pltpu.VMEM_SHARED((64, 2048), jnp.int32),
        ],
        compiler_params=pltpu.CompilerParams(
            use_tc_tiling_on_sc=False,
            needs_layout_passes=False,
        ),
    )
    return f(x3, bf_pad)


def _apply_body(t_ref, bf_ref, x_ref, o_ref):
    xv = x_ref[...]
    boosted = xv * bf_ref[...]
    bits = lax.bitcast_convert_type(boosted, jnp.int32)
    neg = bits >> 31
    key = bits ^ (neg | _MIN_I32)
    key_u = lax.bitcast_convert_type(key, jnp.uint32)
    t_u = lax.bitcast_convert_type(t_ref[0, 0, 0], jnp.uint32)
    o_ref[...] = jnp.where(key_u >= t_u, xv, jnp.float32(0.0))


def kernel(x, duty_cycles):
    B, C, H, W = x.shape
    S = H * W
    bf = jnp.exp(jnp.float32(_K / _N) - duty_cycles.reshape(C))
    bf_pad = jnp.broadcast_to(bf.reshape(C, 1), (C, 16))
    x3 = x.reshape(B, C, S)

    thresholds = _sc_select(x3, bf_pad)

    NS = 8
    SB = S // NS
    out = pl.pallas_call(
        _apply_body,
        grid=(B, NS),
        in_specs=[
            pl.BlockSpec((1, 1, 16), lambda b, j: (b, 0, 0),
                         memory_space=pltpu.SMEM),
            pl.BlockSpec((1, C, 1), lambda b, j: (0, 0, 0)),
            pl.BlockSpec((1, C, SB), lambda b, j: (b, 0, j)),
        ],
        out_specs=pl.BlockSpec((1, C, SB), lambda b, j: (b, 0, j)),
        out_shape=jax.ShapeDtypeStruct((B, C, S), jnp.float32),
    )(thresholds.reshape(B, 1, 16), bf.reshape(1, C, 1), x3)
    return out.reshape(B, C, H, W)


# speedup vs baseline: 3.5037x; 3.5037x over previous
import jax
import jax.numpy as jnp
from jax import lax
from jax.experimental import pallas as pl
from jax.experimental.pallas import tpu as pltpu
from jax.experimental.pallas import tpu_sc as plsc

_B = 8
_C = 96
_S = 224 * 224
_N = _C * _S
_K = int(round(0.1 * _N))

_NCORES = 2
_NSUB = 16
_ROWS_PER_CORE = _B // _NCORES
_TILES_PER_ROW = _NSUB // _ROWS_PER_CORE
_CH_PER_TILE = _C // _TILES_PER_ROW
_CHUNK = _S // 4
_CHUNKS_PER_TILE = _CH_PER_TILE * 4
_VREGS_PER_CHUNK = _CHUNK // 16

_MIN_I32 = -(2 ** 31)
_PASSES = ((21, 11), (10, 11), (0, 10))


def _sc_body(x_hbm, bf_hbm, out_hbm, bf_v, stage0_v, stage1_v, hist_v, idx_v,
             tvec_v, merged_sh, sem0, sem1):
    c = lax.axis_index("c")
    s = lax.axis_index("s")
    local_row = s // _TILES_PER_ROW
    grp = s % _TILES_PER_ROW
    row = c * _ROWS_PER_CORE + local_row
    lane = lax.iota(jnp.int32, 16)
    ones = jnp.ones((16,), jnp.int32)
    zeros16 = jnp.zeros((16,), jnp.int32)

    pltpu.sync_copy(bf_hbm, bf_v)

    def fill_idx(j, _):
        def fill_t(t, _):
            idx_v[j, pl.ds(t * 16, 16)] = (local_row * 2048 + j * 128
                                           + t * 16 + lane)
            return 0
        lax.fori_loop(0, 8, fill_t, 0)
        return 0
    lax.fori_loop(0, 16, fill_idx, 0)

    stages = (stage0_v, stage1_v)
    sems = (sem0, sem1)

    def chunk_src(i):
        ch = grp * _CH_PER_TILE + i // 4
        q = i % 4
        return x_hbm.at[row, ch, pl.ds(q * _CHUNK, _CHUNK)]

    rem = jnp.int32(_K)
    prefix = jnp.int32(0)

    for shift, width in _PASSES:
        nbins = 1 << width
        first = shift == _PASSES[0][0]

        def zbody(i, _):
            hist_v[i, :] = zeros16
            return 0
        lax.fori_loop(0, 2048, zbody, 0)
        pltpu.sync_copy(hist_v.at[pl.ds(0, 512), :],
                        merged_sh.at[pl.ds(s * 512, 512), :])
        plsc.subcore_barrier()

        shift_vec = jnp.full((16,), shift, jnp.int32)
        binmask_vec = jnp.full((16,), nbins - 1, jnp.int32)
        pshift_vec = jnp.full((16,), shift + width, jnp.int32)
        prefix_vec = jnp.full((16,), prefix, jnp.int32)

        pltpu.make_async_copy(chunk_src(0), stage0_v, sem0).start()
        pltpu.make_async_copy(chunk_src(1), stage1_v, sem1).start()

        def chunk_group(g, _):
            for b in range(2):
                i = 2 * g + b
                stage, sem = stages[b], sems[b]
                pltpu.make_async_copy(chunk_src(i), stage, sem).wait()
                ch = grp * _CH_PER_TILE + i // 4
                bfv = bf_v[ch, :]

                @plsc.parallel_loop(0, _VREGS_PER_CHUNK, unroll=8)
                def _(j):
                    v = stage[pl.ds(j * 16, 16)]
                    boosted = v * bfv
                    bits = lax.bitcast_convert_type(boosted, jnp.int32)
                    neg = bits >> 31
                    key = bits ^ (neg | _MIN_I32)
                    bin_ = (lax.shift_right_logical(key, shift_vec)
                            & binmask_vec)
                    if first:
                        plsc.addupdate_scatter(hist_v, [bin_, lane], ones)
                    else:
                        mk = (lax.shift_right_logical(key, pshift_vec)
                              == prefix_vec)
                        plsc.addupdate_scatter(hist_v, [bin_, lane], ones,
                                               mask=mk)

                @pl.when(i + 2 < _CHUNKS_PER_TILE)
                def _():
                    pltpu.make_async_copy(chunk_src(i + 2), stage, sem).start()
            return 0
        lax.fori_loop(0, _CHUNKS_PER_TILE // 2, chunk_group, 0)

        for j in range(16):
            pltpu.sync_copy(hist_v.at[pl.ds(j * 128, 128), :],
                            merged_sh.at[idx_v.at[j]], add=True)
        plsc.subcore_barrier()
        pltpu.sync_copy(merged_sh.at[pl.ds(local_row * 2048, 2048), :], hist_v)
        plsc.subcore_barrier()

        def scan_body(i, carry):
            cum, found, b_sel, rem_next = carry
            b = nbins - 1 - i
            cnt = jnp.sum(hist_v[b, :])
            cum_new = cum + cnt
            crossed = cum_new >= rem
            take = jnp.logical_and(crossed, jnp.logical_not(found))
            b_sel = jnp.where(take, b, b_sel)
            rem_next = jnp.where(take, rem - cum, rem_next)
            found = jnp.logical_or(found, crossed)
            return (cum_new, found, b_sel, rem_next)

        init = (jnp.int32(0), jnp.bool_(False), jnp.int32(0), rem)
        _, _, b_sel, rem = lax.fori_loop(0, nbins, scan_body, init)
        prefix = (prefix << width) | b_sel

    @pl.when(grp == 0)
    def _():
        tvec_v[...] = jnp.full((16,), prefix, jnp.int32)
        pltpu.sync_copy(tvec_v, out_hbm.at[row])


def _sc_select(x3, bf_pad):
    mesh = plsc.VectorSubcoreMesh(core_axis_name="c", subcore_axis_name="s")
    f = pl.kernel(
        _sc_body,
        out_type=jax.ShapeDtypeStruct((_B, 16), jnp.int32),
        mesh=mesh,
        scratch_types=[
            pltpu.VMEM((_C, 16), jnp.float32),
            pltpu.VMEM((_CHUNK,), jnp.float32),
            pltpu.VMEM((_CHUNK,), jnp.float32),
            pltpu.VMEM((2048, 16), jnp.int32),
            pltpu.VMEM((16, 128), jnp.int32),
            pltpu.VMEM((16,), jnp.int32),
            pltpu.VMEM_SHARED((8192, 16), jnp.int32),
            pltpu.SemaphoreType.DMA,
            pltpu.SemaphoreType.DMA,
        ],
        compiler_params=pltpu.CompilerParams(
            use_tc_tiling_on_sc=False,
            needs_layout_passes=False,
        ),
    )
    return f(x3, bf_pad)


def _apply_body(t_ref, bf_ref, x_ref, o_ref):
    xv = x_ref[...]
    boosted = xv * bf_ref[...]
    bits = lax.bitcast_convert_type(boosted, jnp.int32)
    neg = bits >> 31
    key = bits ^ (neg | _MIN_I32)
    key_u = lax.bitcast_convert_type(key, jnp.uint32)
    t_u = lax.bitcast_convert_type(t_ref[0, 0, 0], jnp.uint32)
    o_ref[...] = jnp.where(key_u >= t_u, xv, jnp.float32(0.0))


def kernel(x, duty_cycles):
    B, C, H, W = x.shape
    S = H * W
    bf = jnp.exp(jnp.float32(_K / _N) - duty_cycles.reshape(C))
    bf_pad = jnp.broadcast_to(bf.reshape(C, 1), (C, 16))
    x3 = x.reshape(B, C, S)

    thresholds = _sc_select(x3, bf_pad)

    NS = 8
    SB = S // NS
    out = pl.pallas_call(
        _apply_body,
        grid=(B, NS),
        in_specs=[
            pl.BlockSpec((1, 1, 16), lambda b, j: (b, 0, 0),
                         memory_space=pltpu.SMEM),
            pl.BlockSpec((1, C, 1), lambda b, j: (0, 0, 0)),
            pl.BlockSpec((1, C, SB), lambda b, j: (b, 0, j)),
        ],
        out_specs=pl.BlockSpec((1, C, SB), lambda b, j: (b, 0, j)),
        out_shape=jax.ShapeDtypeStruct((B, C, S), jnp.float32),
    )(thresholds.reshape(B, 1, 16), bf.reshape(1, C, 1), x3)
    return out.reshape(B, C, H, W)
